# baseline (speedup 1.0000x reference)
import functools

import jax
import jax.numpy as jnp
from jax import lax
from jax.experimental import pallas as pl
from jax.experimental.pallas import tpu as pltpu
from jax.experimental.pallas import tpu_sc as plsc

N = 10000
E = 160000
D = 256
H = 256
C = 10
K = 10
ALPHA = 0.1

TILES = 16
CHUNK = 640
N_PAD = TILES * CHUNK
ROW = 128
EC = 80
E_PAD = TILES * EC * ROW


def _rsqrt16(d):
    m = d
    scale = jnp.full((16,), 1.0, jnp.float32)
    for _unused in range(9):
        big = m >= 4.0
        m = jnp.where(big, m * 0.25, m)
        scale = jnp.where(big, scale * 0.5, scale)
    y = jnp.full((16,), 0.6, jnp.float32)
    for _unused in range(5):
        y = y * (1.5 - 0.5 * m * y * y)
    return y * scale


def _sc_body(ei_ref, v_ref, src_i, dst_i, t_buf, un_tab, z_c, norm_c, v_c,
             un_c, zero_c, z_sh, un_sh, sem_s):
    sid = lax.axis_index("s")
    ebase = sid * EC
    cbase = sid * CHUNK
    cslice = pl.ds(cbase, CHUNK)

    pltpu.sync_copy(ei_ref.at[0, pl.ds(ebase, EC)], src_i)
    pltpu.sync_copy(ei_ref.at[1, pl.ds(ebase, EC)], dst_i)

    def _zero(i, _):
        zero_c[pl.ds(i * 16, 16)] = jnp.zeros((16,), jnp.float32)
        return 0

    lax.fori_loop(0, CHUNK // 16, _zero, 0, unroll=4)
    pltpu.sync_copy(zero_c, z_sh.at[cslice])

    def _ones_row(j, _):
        for m in range(ROW // 16):
            t_buf[j, pl.ds(m * 16, 16)] = jnp.ones((16,), jnp.float32)
        return 0

    lax.fori_loop(0, EC, _ones_row, 0)
    plsc.subcore_barrier()

    def _deg(j, _):
        pltpu.async_copy(t_buf.at[j], z_sh.at[dst_i.at[j]], sem_s, add=True)
        return 0

    lax.fori_loop(0, EC, _deg, 0)
    pltpu.make_async_copy(v_ref, un_tab, sem_s).wait()
    plsc.subcore_barrier()

    pltpu.sync_copy(z_sh.at[cslice], z_c)

    def _norm(i, _):
        s = pl.ds(i * 16, 16)
        norm_c[s] = _rsqrt16(jnp.maximum(z_c[s], 1.0))
        v_c[s] = jnp.full((16,), ALPHA, jnp.float32)
        return 0

    lax.fori_loop(0, CHUNK // 16, _norm, 0)
    pltpu.sync_copy(norm_c, un_sh.at[cslice])
    pltpu.sync_copy(zero_c, z_sh.at[cslice])
    plsc.subcore_barrier()

    for step in range(1, K + 1):
        coef = ALPHA * (1.0 - ALPHA) ** step if step < K else (1.0 - ALPHA) ** K

        pltpu.sync_copy(un_sh, un_tab)

        def _edge(j, _):
            for m in range(ROW // 16):
                sl = pl.ds(m * 16, 16)
                t_buf[j, sl] = plsc.load_gather(un_tab, [dst_i[j, sl]])
            pltpu.async_copy(t_buf.at[j], z_sh.at[src_i.at[j]], sem_s,
                             add=True)
            return 0

        lax.fori_loop(0, EC, _edge, 0, unroll=2)
        pltpu.make_async_copy(v_ref, un_tab, sem_s).wait()
        plsc.subcore_barrier()

        pltpu.sync_copy(z_sh.at[cslice], z_c)

        def _upd(i, _):
            s = pl.ds(i * 16, 16)
            u = z_c[s] * norm_c[s]
            v_c[s] = v_c[s] + coef * u
            un_c[s] = u * norm_c[s]
            return 0

        lax.fori_loop(0, CHUNK // 16, _upd, 0, unroll=4)
        if step < K:
            pltpu.sync_copy(un_c, un_sh.at[cslice])
            pltpu.sync_copy(zero_c, z_sh.at[cslice])
            plsc.subcore_barrier()

    def _mask(i, _):
        s = pl.ds(i * 16, 16)
        ii = lax.iota(jnp.int32, 16) + (cbase + i * 16)
        v_c[s] = jnp.where(ii < N, v_c[s], 0.0)
        return 0

    lax.fori_loop(0, CHUNK // 16, _mask, 0)
    pltpu.sync_copy(v_c, v_ref.at[cslice])


_sc_propagate = functools.partial(
    pl.kernel,
    out_type=jax.ShapeDtypeStruct((N_PAD,), jnp.float32),
    mesh=plsc.VectorSubcoreMesh(core_axis_name="c", subcore_axis_name="s",
                                num_cores=1),
    compiler_params=pltpu.CompilerParams(needs_layout_passes=False),
    scratch_types=[
        pltpu.VMEM((EC, ROW), jnp.int32),
        pltpu.VMEM((EC, ROW), jnp.int32),
        pltpu.VMEM((EC, ROW), jnp.float32),
        pltpu.VMEM((N_PAD,), jnp.float32),
        pltpu.VMEM((CHUNK,), jnp.float32),
        pltpu.VMEM((CHUNK,), jnp.float32),
        pltpu.VMEM((CHUNK,), jnp.float32),
        pltpu.VMEM((CHUNK,), jnp.float32),
        pltpu.VMEM((CHUNK,), jnp.float32),
        pltpu.VMEM_SHARED((N_PAD,), jnp.float32),
        pltpu.VMEM_SHARED((N_PAD,), jnp.float32),
        pltpu.SemaphoreType.DMA,
    ],
)(_sc_body)


TILE_M = 1024
GRID_M = N_PAD // TILE_M


def _tc_mlp_body(x_ref, w0_ref, b0_ref, w1_ref, b1_ref, w2_ref, b2_ref,
                 wc_ref, g_ref):
    h = jnp.maximum(
        jnp.dot(x_ref[...], w0_ref[...], preferred_element_type=jnp.float32)
        + b0_ref[...], 0.0)
    h = jnp.maximum(
        jnp.dot(h, w1_ref[...], preferred_element_type=jnp.float32)
        + b1_ref[...], 0.0)
    h = jnp.maximum(
        jnp.dot(h, w2_ref[...], preferred_element_type=jnp.float32)
        + b2_ref[...], 0.0)
    g_ref[...] = jnp.dot(h, wc_ref[...], preferred_element_type=jnp.float32)


_tc_mlp = pl.pallas_call(
    _tc_mlp_body,
    grid=(GRID_M,),
    in_specs=[
        pl.BlockSpec((TILE_M, D), lambda i: (i, 0)),
        pl.BlockSpec((D, H), lambda i: (0, 0)),
        pl.BlockSpec((1, H), lambda i: (0, 0)),
        pl.BlockSpec((H, H), lambda i: (0, 0)),
        pl.BlockSpec((1, H), lambda i: (0, 0)),
        pl.BlockSpec((H, H), lambda i: (0, 0)),
        pl.BlockSpec((1, H), lambda i: (0, 0)),
        pl.BlockSpec((H, C), lambda i: (0, 0)),
    ],
    out_specs=pl.BlockSpec((TILE_M, C), lambda i: (i, 0)),
    out_shape=jax.ShapeDtypeStruct((N_PAD, C), jnp.float32),
    compiler_params=pltpu.CompilerParams(
        dimension_semantics=("arbitrary",)),
)


def _tc_out_body(v_ref, g_ref, bc_ref, out_ref):
    out_ref[...] = jnp.dot(
        v_ref[...] * (1.0 / N), g_ref[...],
        preferred_element_type=jnp.float32) + bc_ref[...]


_tc_readout = pl.pallas_call(
    _tc_out_body,
    grid=(1,),
    in_specs=[
        pl.BlockSpec((1, N_PAD), lambda i: (0, 0)),
        pl.BlockSpec((N_PAD, C), lambda i: (0, 0)),
        pl.BlockSpec((1, C), lambda i: (0, 0)),
    ],
    out_specs=pl.BlockSpec((1, C), lambda i: (0, 0)),
    out_shape=jax.ShapeDtypeStruct((1, C), jnp.float32),
)


def kernel(x, edge_index, W0, b0, W1, b1, W2, b2, Wc, bc):
    ei = jnp.pad(edge_index, ((0, 0), (0, E_PAD - E)), constant_values=N)
    ei3 = ei.reshape(2, TILES * EC, ROW)
    xp = jnp.pad(x, ((0, N_PAD - N), (0, 0)))
    g = _tc_mlp(xp, W0, b0.reshape(1, H), W1, b1.reshape(1, H),
                W2, b2.reshape(1, H), Wc)
    v = _sc_propagate(ei3)
    return _tc_readout(v.reshape(1, N_PAD), g, bc.reshape(1, C))


# speedup vs baseline: 1.0981x; 1.0981x over previous
import functools

import jax
import jax.numpy as jnp
from jax import lax
from jax.experimental import pallas as pl
from jax.experimental.pallas import tpu as pltpu
from jax.experimental.pallas import tpu_sc as plsc

N = 10000
E = 160000
D = 256
H = 256
C = 10
K = 10
ALPHA = 0.1

TILES = 16
CHUNK = 640
N_PAD = TILES * CHUNK
ROW = 128
EC = 80
E_PAD = TILES * EC * ROW


def _rsqrt16(d):
    m = d
    scale = jnp.full((16,), 1.0, jnp.float32)
    for _unused in range(9):
        big = m >= 4.0
        m = jnp.where(big, m * 0.25, m)
        scale = jnp.where(big, scale * 0.5, scale)
    y = jnp.full((16,), 0.6, jnp.float32)
    for _unused in range(5):
        y = y * (1.5 - 0.5 * m * y * y)
    return y * scale


def _sc_body(ei_ref, v_ref, src_i, dst_i, t_buf, un_tab, z_c, norm_c, v_c,
             un_c, zero_c, z_sh, un_sh, sem_s):
    sid = lax.axis_index("s")
    ebase = sid * EC
    cbase = sid * CHUNK
    cslice = pl.ds(cbase, CHUNK)

    pltpu.sync_copy(ei_ref.at[0, pl.ds(ebase, EC)], src_i)
    pltpu.sync_copy(ei_ref.at[1, pl.ds(ebase, EC)], dst_i)

    def _zero(i, _):
        zero_c[pl.ds(i * 16, 16)] = jnp.zeros((16,), jnp.float32)
        return 0

    lax.fori_loop(0, CHUNK // 16, _zero, 0, unroll=4)
    pltpu.sync_copy(zero_c, z_sh.at[cslice])

    def _ones_row(j, _):
        for m in range(ROW // 16):
            t_buf[j, pl.ds(m * 16, 16)] = jnp.ones((16,), jnp.float32)
        return 0

    lax.fori_loop(0, EC, _ones_row, 0)
    plsc.subcore_barrier()

    def _deg(j, _):
        pltpu.async_copy(t_buf.at[j], z_sh.at[dst_i.at[j]], sem_s, add=True)
        return 0

    lax.fori_loop(0, EC, _deg, 0)
    pltpu.make_async_copy(v_ref, un_tab, sem_s).wait()
    plsc.subcore_barrier()

    pltpu.sync_copy(z_sh.at[cslice], z_c)

    def _norm(i, _):
        s = pl.ds(i * 16, 16)
        norm_c[s] = _rsqrt16(jnp.maximum(z_c[s], 1.0))
        v_c[s] = jnp.full((16,), ALPHA, jnp.float32)
        return 0

    lax.fori_loop(0, CHUNK // 16, _norm, 0)
    pltpu.sync_copy(norm_c, un_sh.at[cslice])
    pltpu.sync_copy(zero_c, z_sh.at[cslice])
    plsc.subcore_barrier()

    for step in range(1, K + 1):
        coef = ALPHA * (1.0 - ALPHA) ** step if step < K else (1.0 - ALPHA) ** K

        pltpu.sync_copy(un_sh, un_tab)

        def _edge(j, _):
            for m in range(ROW // 16):
                sl = pl.ds(m * 16, 16)
                t_buf[j, sl] = plsc.load_gather(un_tab, [dst_i[j, sl]])
            pltpu.async_copy(t_buf.at[j], z_sh.at[src_i.at[j]], sem_s,
                             add=True)
            return 0

        lax.fori_loop(0, EC, _edge, 0)
        pltpu.make_async_copy(v_ref, un_tab, sem_s).wait()
        plsc.subcore_barrier()

        pltpu.sync_copy(z_sh.at[cslice], z_c)

        def _upd(i, _):
            s = pl.ds(i * 16, 16)
            u = z_c[s] * norm_c[s]
            v_c[s] = v_c[s] + coef * u
            un_c[s] = u * norm_c[s]
            return 0

        lax.fori_loop(0, CHUNK // 16, _upd, 0, unroll=4)
        if step < K:
            pltpu.sync_copy(un_c, un_sh.at[cslice])
            pltpu.sync_copy(zero_c, z_sh.at[cslice])
            plsc.subcore_barrier()

    def _mask(i, _):
        s = pl.ds(i * 16, 16)
        ii = lax.iota(jnp.int32, 16) + (cbase + i * 16)
        v_c[s] = jnp.where(ii < N, v_c[s], 0.0)
        return 0

    lax.fori_loop(0, CHUNK // 16, _mask, 0)
    pltpu.sync_copy(v_c, v_ref.at[cslice])


_sc_propagate = functools.partial(
    pl.kernel,
    out_type=jax.ShapeDtypeStruct((N_PAD,), jnp.float32),
    mesh=plsc.VectorSubcoreMesh(core_axis_name="c", subcore_axis_name="s",
                                num_cores=1),
    compiler_params=pltpu.CompilerParams(needs_layout_passes=False),
    scratch_types=[
        pltpu.VMEM((EC, ROW), jnp.int32),
        pltpu.VMEM((EC, ROW), jnp.int32),
        pltpu.VMEM((EC, ROW), jnp.float32),
        pltpu.VMEM((N_PAD,), jnp.float32),
        pltpu.VMEM((CHUNK,), jnp.float32),
        pltpu.VMEM((CHUNK,), jnp.float32),
        pltpu.VMEM((CHUNK,), jnp.float32),
        pltpu.VMEM((CHUNK,), jnp.float32),
        pltpu.VMEM((CHUNK,), jnp.float32),
        pltpu.VMEM_SHARED((N_PAD,), jnp.float32),
        pltpu.VMEM_SHARED((N_PAD,), jnp.float32),
        pltpu.SemaphoreType.DMA,
    ],
)(_sc_body)


TILE_M = 1024
GRID_M = N_PAD // TILE_M


def _tc_mlp_body(x_ref, w0_ref, b0_ref, w1_ref, b1_ref, w2_ref, b2_ref,
                 wc_ref, g_ref):
    h = jnp.maximum(
        jnp.dot(x_ref[...], w0_ref[...], preferred_element_type=jnp.float32)
        + b0_ref[...], 0.0)
    h = jnp.maximum(
        jnp.dot(h, w1_ref[...], preferred_element_type=jnp.float32)
        + b1_ref[...], 0.0)
    h = jnp.maximum(
        jnp.dot(h, w2_ref[...], preferred_element_type=jnp.float32)
        + b2_ref[...], 0.0)
    g_ref[...] = jnp.dot(h, wc_ref[...], preferred_element_type=jnp.float32)


_tc_mlp = pl.pallas_call(
    _tc_mlp_body,
    grid=(GRID_M,),
    in_specs=[
        pl.BlockSpec((TILE_M, D), lambda i: (i, 0)),
        pl.BlockSpec((D, H), lambda i: (0, 0)),
        pl.BlockSpec((1, H), lambda i: (0, 0)),
        pl.BlockSpec((H, H), lambda i: (0, 0)),
        pl.BlockSpec((1, H), lambda i: (0, 0)),
        pl.BlockSpec((H, H), lambda i: (0, 0)),
        pl.BlockSpec((1, H), lambda i: (0, 0)),
        pl.BlockSpec((H, C), lambda i: (0, 0)),
    ],
    out_specs=pl.BlockSpec((TILE_M, C), lambda i: (i, 0)),
    out_shape=jax.ShapeDtypeStruct((N_PAD, C), jnp.float32),
    compiler_params=pltpu.CompilerParams(
        dimension_semantics=("arbitrary",)),
)


def _tc_out_body(v_ref, g_ref, bc_ref, out_ref):
    out_ref[...] = jnp.dot(
        v_ref[...] * (1.0 / N), g_ref[...],
        preferred_element_type=jnp.float32) + bc_ref[...]


_tc_readout = pl.pallas_call(
    _tc_out_body,
    grid=(1,),
    in_specs=[
        pl.BlockSpec((1, N_PAD), lambda i: (0, 0)),
        pl.BlockSpec((N_PAD, C), lambda i: (0, 0)),
        pl.BlockSpec((1, C), lambda i: (0, 0)),
    ],
    out_specs=pl.BlockSpec((1, C), lambda i: (0, 0)),
    out_shape=jax.ShapeDtypeStruct((1, C), jnp.float32),
)


def kernel(x, edge_index, W0, b0, W1, b1, W2, b2, Wc, bc):
    ei = jnp.pad(edge_index, ((0, 0), (0, E_PAD - E)), constant_values=N)
    ei3 = ei.reshape(2, TILES * EC, ROW)
    xp = jnp.pad(x, ((0, N_PAD - N), (0, 0)))
    g = _tc_mlp(xp, W0, b0.reshape(1, H), W1, b1.reshape(1, H),
                W2, b2.reshape(1, H), Wc)
    v = _sc_propagate(ei3)
    return _tc_readout(v.reshape(1, N_PAD), g, bc.reshape(1, C))
